# baseline (speedup 1.0000x reference)
import functools

import jax
import jax.numpy as jnp
from jax import lax
from jax.experimental import pallas as pl
from jax.experimental.pallas import tpu as pltpu
from jax.experimental.pallas import tpu_sc as plsc

_TOKENS = 16384
_D = 2048
_E = 64
_BT = 2048

_NC = 2
_NS = 16
_LANES = 16
_TPW = _TOKENS // (_NC * _NS)


def _proj_softmax_body(x_ref, wt_ref, probs_ref):
    logits = jax.lax.dot_general(
        x_ref[...], wt_ref[...], (((1,), (0,)), ((), ())),
        preferred_element_type=jnp.float32,
        precision=jax.lax.Precision.DEFAULT,
    )
    m = jnp.max(logits, axis=-1, keepdims=True)
    e = jnp.exp(logits - m)
    probs_ref[...] = e / jnp.sum(e, axis=-1, keepdims=True)


def _tc_probs(x, wt):
    return pl.pallas_call(
        _proj_softmax_body,
        grid=(_TOKENS // _BT,),
        in_specs=[
            pl.BlockSpec((_BT, _D), lambda i: (i, 0)),
            pl.BlockSpec((_D, _E), lambda i: (0, 0)),
        ],
        out_specs=pl.BlockSpec((_BT, _E), lambda i: (i, 0)),
        out_shape=jax.ShapeDtypeStruct((_TOKENS, _E), jnp.float32),
    )(x, wt)


@functools.partial(
    pl.kernel,
    mesh=plsc.VectorSubcoreMesh(core_axis_name="c", subcore_axis_name="s"),
    compiler_params=pltpu.CompilerParams(needs_layout_passes=False),
    out_type=[
        jax.ShapeDtypeStruct((_TOKENS * 2,), jnp.float32),
        jax.ShapeDtypeStruct((_TOKENS * 2,), jnp.int32),
    ],
    scratch_types=[
        pltpu.VMEM((_TPW * _E,), jnp.float32),
        pltpu.VMEM((_TPW * 2,), jnp.float32),
        pltpu.VMEM((_TPW * 2,), jnp.int32),
    ],
)
def _sc_top2(probs_hbm, gates_hbm, idx_hbm, probs_v, gates_v, idx_v):
    wid = lax.axis_index("s") * _NC + lax.axis_index("c")
    pltpu.sync_copy(probs_hbm.at[pl.ds(wid * (_TPW * _E), _TPW * _E)], probs_v)

    def merge(a, b):
        a1, ai1, a2, ai2 = a
        b1, bi1, b2, bi2 = b
        t = a1 >= b1
        m1 = jnp.where(t, a1, b1)
        i1 = jnp.where(t, ai1, bi1)
        c = jnp.where(t, a2, a1)
        ci = jnp.where(t, ai2, ai1)
        d = jnp.where(t, b1, b2)
        di = jnp.where(t, bi1, bi2)
        u = c >= d
        return (m1, i1, jnp.where(u, c, d), jnp.where(u, ci, di))

    def group(g, carry):
        lanes = lax.iota(jnp.int32, _LANES)
        prow = (g * _LANES + lanes) * _E
        chains = []
        for q in range(4):
            m1 = jnp.full((_LANES,), -1.0, jnp.float32)
            m2 = jnp.full((_LANES,), -1.0, jnp.float32)
            i1 = jnp.zeros((_LANES,), jnp.int32)
            i2 = jnp.zeros((_LANES,), jnp.int32)
            for e in range(16 * q, 16 * q + 16):
                v = plsc.load_gather(probs_v, [prow + e])
                gt1 = v > m1
                gt2 = v > m2
                m2 = jnp.where(gt1, m1, jnp.where(gt2, v, m2))
                i2 = jnp.where(gt1, i1, jnp.where(gt2, e, i2))
                m1 = jnp.where(gt1, v, m1)
                i1 = jnp.where(gt1, e, i1)
            chains.append((m1, i1, m2, i2))
        m1, i1, m2, i2 = merge(merge(chains[0], chains[1]),
                               merge(chains[2], chains[3]))
        s = m1 + m2
        orow = (g * _LANES + lanes) * 2
        plsc.store_scatter(gates_v, [orow], m1 / s)
        plsc.store_scatter(gates_v, [orow + 1], m2 / s)
        plsc.store_scatter(idx_v, [orow], i1)
        plsc.store_scatter(idx_v, [orow + 1], i2)
        return carry

    lax.fori_loop(0, _TPW // _LANES, group, 0)

    pltpu.sync_copy(gates_v, gates_hbm.at[pl.ds(wid * (_TPW * 2), _TPW * 2)])
    pltpu.sync_copy(idx_v, idx_hbm.at[pl.ds(wid * (_TPW * 2), _TPW * 2)])


def kernel(x, W_r):
    probs = _tc_probs(x, W_r.T)
    gates_flat, idx_flat = _sc_top2(probs.reshape(-1))
    return (gates_flat.reshape(_TOKENS, 2), idx_flat.reshape(_TOKENS, 2), probs)


# speedup vs baseline: 1.6990x; 1.6990x over previous
import jax
import jax.numpy as jnp
from jax.experimental import pallas as pl

_TOKENS = 16384
_D = 2048
_E = 64
_BT = 2048


def _router_body(x_ref, wt_ref, probs_ref, gates_ref, idx_ref):
    x = x_ref[...]
    wt = wt_ref[...]
    logits = jax.lax.dot_general(
        x, wt, (((1,), (0,)), ((), ())),
        preferred_element_type=jnp.float32,
        precision=jax.lax.Precision.DEFAULT,
    )
    m = jnp.max(logits, axis=-1, keepdims=True)
    e = jnp.exp(logits - m)
    probs = e / jnp.sum(e, axis=-1, keepdims=True)
    probs_ref[...] = probs

    lane = jax.lax.broadcasted_iota(jnp.int32, probs.shape, 1)
    i1 = jnp.argmax(probs, axis=-1, keepdims=True)
    m1 = jnp.max(probs, axis=-1, keepdims=True)
    masked = jnp.where(lane == i1, -1.0, probs)
    i2 = jnp.argmax(masked, axis=-1, keepdims=True)
    m2 = jnp.max(masked, axis=-1, keepdims=True)
    s = m1 + m2
    gates_ref[...] = jnp.concatenate([m1 / s, m2 / s], axis=1)
    idx_ref[...] = jnp.concatenate([i1, i2], axis=1)


def kernel(x, W_r):
    wt = W_r.T
    grid = (_TOKENS // _BT,)
    probs, gates, idx = pl.pallas_call(
        _router_body,
        grid=grid,
        in_specs=[
            pl.BlockSpec((_BT, _D), lambda i: (i, 0)),
            pl.BlockSpec((_D, _E), lambda i: (0, 0)),
        ],
        out_specs=[
            pl.BlockSpec((_BT, _E), lambda i: (i, 0)),
            pl.BlockSpec((_BT, 2), lambda i: (i, 0)),
            pl.BlockSpec((_BT, 2), lambda i: (i, 0)),
        ],
        out_shape=[
            jax.ShapeDtypeStruct((_TOKENS, _E), jnp.float32),
            jax.ShapeDtypeStruct((_TOKENS, 2), jnp.float32),
            jax.ShapeDtypeStruct((_TOKENS, 2), jnp.int32),
        ],
    )(x, wt)
    return (gates, idx, probs)
